# baseline (speedup 1.0000x reference)
import functools

import jax
import jax.numpy as jnp
from jax import lax
from jax.experimental import pallas as pl
from jax.experimental.pallas import tpu as pltpu
from jax.experimental.pallas import tpu_sc as plsc

_VOCAB = 32002
_HIDDEN = 2048
_BATCH = 4
_SEQ = 4096
_NTOK = _BATCH * _SEQ
_NW = 32
_PER_W = _NTOK // _NW
_CHUNK = 24
_NSLOT = 22
_TAIL = _PER_W - 21 * _CHUNK

_mesh = plsc.VectorSubcoreMesh(core_axis_name="c", subcore_axis_name="s")


@functools.partial(
    pl.kernel,
    out_type=jax.ShapeDtypeStruct((_NTOK, _HIDDEN), jnp.float32),
    mesh=_mesh,
    scratch_types=[
        pltpu.VMEM((_NSLOT, _CHUNK), jnp.int32),
        pltpu.VMEM((_CHUNK, _HIDDEN), jnp.float32),
        pltpu.VMEM((_CHUNK, _HIDDEN), jnp.float32),
        pltpu.SemaphoreType.DMA,
        pltpu.SemaphoreType.DMA,
        pltpu.SemaphoreType.DMA,
        pltpu.SemaphoreType.DMA,
    ],
)
def _embed_lookup(table_hbm, idx_hbm, out_hbm, idx_v, buf0, buf1,
                  g0, g1, o0, o1):
    wid = lax.axis_index("s") * 2 + lax.axis_index("c")
    base = wid * _PER_W
    pltpu.sync_copy(idx_hbm.at[wid], idx_v)

    bufs = (buf0, buf1)
    gsems = (g0, g1)
    osems = (o0, o1)

    def gather_start(j, b):
        pltpu.async_copy(table_hbm.at[idx_v.at[j]], bufs[b], gsems[b])

    def gather_wait(b):
        pltpu.make_async_copy(
            table_hbm.at[idx_v.at[0]], bufs[b], gsems[b]).wait()

    def out_start(j, b):
        pltpu.async_copy(
            bufs[b], out_hbm.at[pl.ds(base + j * _CHUNK, _CHUNK)], osems[b])

    def out_wait(b):
        pltpu.make_async_copy(
            bufs[b], out_hbm.at[pl.ds(base, _CHUNK)], osems[b]).wait()

    for j in range(_NSLOT):
        b = j % 2
        if j >= 2:
            out_wait(b)
        gather_start(j, b)
        if j >= 1:
            bp = (j - 1) % 2
            gather_wait(bp)
            out_start(j - 1, bp)

    gather_wait((_NSLOT - 1) % 2)
    pltpu.async_copy(
        bufs[(_NSLOT - 1) % 2].at[pl.ds(0, _TAIL)],
        out_hbm.at[pl.ds(base + (_NSLOT - 1) * _CHUNK, _TAIL)],
        osems[(_NSLOT - 1) % 2])
    out_wait((_NSLOT - 2) % 2)
    pltpu.make_async_copy(
        bufs[(_NSLOT - 1) % 2].at[pl.ds(0, _TAIL)],
        out_hbm.at[pl.ds(base, _TAIL)],
        osems[(_NSLOT - 1) % 2]).wait()


def kernel(input_ids, embedding_table):
    ids = input_ids.reshape(_NW, _PER_W)
    pad = _NSLOT * _CHUNK - _PER_W
    ids = jnp.concatenate([ids, ids[:, -pad:]], axis=1)
    ids = ids.reshape(_NW, _NSLOT, _CHUNK)
    out = _embed_lookup(embedding_table, ids)
    return out.reshape(_BATCH, _SEQ, _HIDDEN)


# speedup vs baseline: 1.0311x; 1.0311x over previous
import functools

import jax
import jax.numpy as jnp
from jax import lax
from jax.experimental import pallas as pl
from jax.experimental.pallas import tpu as pltpu
from jax.experimental.pallas import tpu_sc as plsc

_VOCAB = 32002
_HIDDEN = 2048
_BATCH = 4
_SEQ = 4096
_NTOK = _BATCH * _SEQ
_NW = 32
_PER_W = _NTOK // _NW
_CHUNK = 8
_NCHUNK = _PER_W // _CHUNK

_mesh = plsc.VectorSubcoreMesh(core_axis_name="c", subcore_axis_name="s")


@functools.partial(
    pl.kernel,
    out_type=jax.ShapeDtypeStruct((_NTOK, _HIDDEN), jnp.float32),
    mesh=_mesh,
    scratch_types=[
        pltpu.VMEM((_NCHUNK, _CHUNK), jnp.int32),
        pltpu.VMEM((_CHUNK, _HIDDEN), jnp.float32),
        pltpu.VMEM((_CHUNK, _HIDDEN), jnp.float32),
        pltpu.VMEM_SHARED((16, 2, _CHUNK, _HIDDEN), jnp.float32),
        pltpu.SemaphoreType.DMA,
        pltpu.SemaphoreType.DMA,
        pltpu.SemaphoreType.DMA,
        pltpu.SemaphoreType.DMA,
        pltpu.SemaphoreType.DMA,
        pltpu.SemaphoreType.DMA,
    ],
)
def _embed_lookup(table_hbm, idx_hbm, out_hbm, idx_v, buf0, buf1, shared,
                  g0, g1, x0, x1, o0, o1):
    sid = lax.axis_index("s")
    wid = sid * 2 + lax.axis_index("c")
    base = wid * _PER_W
    pltpu.sync_copy(idx_hbm.at[wid], idx_v)

    bufs = (buf0, buf1)
    gsems = (g0, g1)
    xsems = (x0, x1)
    osems = (o0, o1)

    def gather_start(j, b):
        pltpu.async_copy(table_hbm.at[idx_v.at[j]], bufs[b], gsems[b])

    def gather_wait(b):
        pltpu.make_async_copy(
            table_hbm.at[idx_v.at[0]], bufs[b], gsems[b]).wait()

    def stage(b):
        pltpu.async_copy(bufs[b], shared.at[sid, b], xsems[b]).wait()

    def out_start(j, b):
        pltpu.async_copy(
            shared.at[sid, b],
            out_hbm.at[pl.ds(base + j * _CHUNK, _CHUNK)], osems[b])

    def out_wait(b):
        pltpu.make_async_copy(
            shared.at[sid, b],
            out_hbm.at[pl.ds(base, _CHUNK)], osems[b]).wait()

    gather_start(0, 0)
    gather_start(1, 1)
    for b in range(2):
        gather_wait(b)
        stage(b)
        out_start(b, b)
        gather_start(b + 2, b)

    def step(k, carry):
        for b in range(2):
            j = 2 * k + b
            out_wait(b)
            gather_wait(b)
            stage(b)
            out_start(j, b)
            gather_start(j + 2, b)
        return carry

    lax.fori_loop(1, _NCHUNK // 2 - 1, step, 0)

    for b in range(2):
        j = _NCHUNK - 2 + b
        out_wait(b)
        gather_wait(b)
        stage(b)
        out_start(j, b)
    out_wait(0)
    out_wait(1)


def kernel(input_ids, embedding_table):
    ids = input_ids.reshape(_NW, _NCHUNK, _CHUNK)
    out = _embed_lookup(embedding_table, ids)
    return out.reshape(_BATCH, _SEQ, _HIDDEN)


# speedup vs baseline: 1.0558x; 1.0239x over previous
import functools

import jax
import jax.numpy as jnp
from jax import lax
from jax.experimental import pallas as pl
from jax.experimental.pallas import tpu as pltpu
from jax.experimental.pallas import tpu_sc as plsc

_VOCAB = 32002
_HIDDEN = 2048
_BATCH = 4
_SEQ = 4096
_NTOK = _BATCH * _SEQ
_NW = 32
_PER_W = _NTOK // _NW
_CHUNK = 8
_NCHUNK = _PER_W // _CHUNK
_NBUF = 3

_mesh = plsc.VectorSubcoreMesh(core_axis_name="c", subcore_axis_name="s")


@functools.partial(
    pl.kernel,
    out_type=jax.ShapeDtypeStruct((_NTOK, _HIDDEN), jnp.float32),
    mesh=_mesh,
    scratch_types=(
        [pltpu.VMEM((_NCHUNK, _CHUNK), jnp.int32)]
        + [pltpu.VMEM((_CHUNK, _HIDDEN), jnp.float32)] * _NBUF
        + [pltpu.VMEM_SHARED((16, _NBUF, _CHUNK, _HIDDEN), jnp.float32)]
        + [pltpu.SemaphoreType.DMA] * (3 * _NBUF)
    ),
)
def _embed_lookup(table_hbm, idx_hbm, out_hbm, idx_v, *scratch):
    sid = lax.axis_index("s")
    wid = sid * 2 + lax.axis_index("c")
    base = wid * _PER_W
    pltpu.sync_copy(idx_hbm.at[wid], idx_v)

    bufs = scratch[:_NBUF]
    shared = scratch[_NBUF]
    gsems = scratch[_NBUF + 1:2 * _NBUF + 1]
    xsems = scratch[2 * _NBUF + 1:3 * _NBUF + 1]
    osems = scratch[3 * _NBUF + 1:]

    def gather_start(j, b):
        pltpu.async_copy(table_hbm.at[idx_v.at[j]], bufs[b], gsems[b])

    def gather_wait(b):
        pltpu.make_async_copy(
            table_hbm.at[idx_v.at[0]], bufs[b], gsems[b]).wait()

    def stage(b):
        pltpu.async_copy(bufs[b], shared.at[sid, b], xsems[b]).wait()

    def out_start(j, b):
        pltpu.async_copy(
            shared.at[sid, b],
            out_hbm.at[pl.ds(base + j * _CHUNK, _CHUNK)], osems[b])

    def out_wait(b):
        pltpu.make_async_copy(
            shared.at[sid, b],
            out_hbm.at[pl.ds(base, _CHUNK)], osems[b]).wait()

    def slot_body(j, b, first=False, last=False):
        if not first:
            out_wait(b)
        gather_wait(b)
        stage(b)
        out_start(j, b)
        if not last:
            gather_start(j + _NBUF, b)

    for b in range(_NBUF):
        gather_start(b, b)
    for b in range(_NBUF):
        slot_body(b, b, first=True)

    def step(k, carry):
        for p in range(_NBUF):
            slot_body(_NBUF * k + p, p)
        return carry

    _KMAX = (_NCHUNK - _NBUF) // _NBUF - 1
    lax.fori_loop(1, _KMAX + 1, step, 0)

    _TAIL_START = _NBUF * (_KMAX + 1)
    slot_body(_TAIL_START, _TAIL_START % _NBUF)
    for j in range(_TAIL_START + 1, _NCHUNK):
        slot_body(j, j % _NBUF, last=True)
    for j in range(_NCHUNK - _NBUF, _NCHUNK):
        out_wait(j % _NBUF)


def kernel(input_ids, embedding_table):
    ids = input_ids.reshape(_NW, _NCHUNK, _CHUNK)
    out = _embed_lookup(embedding_table, ids)
    return out.reshape(_BATCH, _SEQ, _HIDDEN)
